# baseline (speedup 1.0000x reference)
import functools

import jax
import jax.numpy as jnp
from jax import lax
from jax.experimental import pallas as pl
from jax.experimental.pallas import tpu as pltpu
from jax.experimental.pallas import tpu_sc as plsc

NC = 2
NS = 16
NW = NC * NS
L = 16

NX = 4096
NY = 16384
C = 128
NB = 8
PF = NY // NW
NG = PF // L
NCHUNK = NX // L

_INF = float("inf")


def _splat(v, t):
  return v[jnp.full((L,), t, dtype=jnp.int32)]


def _knn_gather_sc(xs, ys, zs, bx, fx, fy, fz, fb, xfeat):
  mesh = plsc.VectorSubcoreMesh(core_axis_name="c", subcore_axis_name="s")

  @functools.partial(
      pl.kernel,
      mesh=mesh,
      out_type=jax.ShapeDtypeStruct((NY, C), jnp.float32),
      scratch_types=[
          pltpu.VMEM((NX,), jnp.float32),
          pltpu.VMEM((NX,), jnp.float32),
          pltpu.VMEM((NX,), jnp.float32),
          pltpu.VMEM((NX,), jnp.int32),
          pltpu.VMEM((PF,), jnp.float32),
          pltpu.VMEM((PF,), jnp.float32),
          pltpu.VMEM((PF,), jnp.float32),
          pltpu.VMEM((PF,), jnp.int32),
          pltpu.SMEM((L,), jnp.int32),
          pltpu.VMEM((2, 6 * L), jnp.int32),
          pltpu.VMEM((2, 6 * L, C), jnp.float32),
          pltpu.VMEM((PF, C), jnp.float32),
          pltpu.SemaphoreType.DMA,
          pltpu.SemaphoreType.DMA,
      ],
  )
  def body(xs_h, ys_h, zs_h, bx_h, fx_h, fy_h, fz_h, fb_h, xf_h, out_h,
           xs_v, ys_v, zs_v, bx_v, fx_v, fy_v, fz_v, fb_v, st_s, idx_v,
           rows_v, xi_all, sem0, sem1):
    wid = lax.axis_index("s") * NC + lax.axis_index("c")
    base = pl.multiple_of(wid << 9, 512)

    pltpu.sync_copy(xs_h, xs_v)
    pltpu.sync_copy(ys_h, ys_v)
    pltpu.sync_copy(zs_h, zs_v)
    pltpu.sync_copy(bx_h, bx_v)
    pltpu.sync_copy(fx_h.at[pl.ds(base, PF)], fx_v)
    pltpu.sync_copy(fy_h.at[pl.ds(base, PF)], fy_v)
    pltpu.sync_copy(fz_h.at[pl.ds(base, PF)], fz_v)
    pltpu.sync_copy(fb_h.at[pl.ds(base, PF)], fb_v)

    iota = lax.iota(jnp.int32, L)

    def _hist(c, cnt):
      bs_c = bx_v[pl.ds(c << 4, L)]
      for t in range(L):
        s = _splat(bs_c, t)
        cnt = cnt + jnp.where(s < iota, 1, 0).astype(jnp.int32)
      return cnt

    starts = lax.fori_loop(0, NCHUNK, _hist, jnp.zeros((L,), jnp.int32))
    for b in range(NB + 1):
      st_s[b] = starts[b]

    def _insert(d, jv, t1, t2, t3, i1, i2, i3):
      c1 = d < t1
      c2 = d < t2
      c3 = d < t3
      nt3 = jnp.where(c2, t2, jnp.where(c3, d, t3))
      ni3 = jnp.where(c2, i2, jnp.where(c3, jv, i3))
      nt2 = jnp.where(c1, t1, jnp.where(c2, d, t2))
      ni2 = jnp.where(c1, i1, jnp.where(c2, jv, i2))
      nt1 = jnp.where(c1, d, t1)
      ni1 = jnp.where(c1, jv, i1)
      return nt1, nt2, nt3, ni1, ni2, ni3

    def _knn_start(p):
      pg = p & 1
      goff = pl.multiple_of(p << 5, 32)
      yxa = fx_v[pl.ds(goff, L)]
      yya = fy_v[pl.ds(goff, L)]
      yza = fz_v[pl.ds(goff, L)]
      yba = fb_v[pl.ds(goff, L)]
      yxb = fx_v[pl.ds(goff + L, L)]
      yyb = fy_v[pl.ds(goff + L, L)]
      yzb = fz_v[pl.ds(goff + L, L)]
      ybb = fb_v[pl.ds(goff + L, L)]

      b_lo = yba[0]
      b_hi = ybb[15]
      j_lo = st_s[b_lo]
      j_hi = st_s[b_hi + 1]
      c_lo = j_lo >> 4
      c_hi = (j_hi + (L - 1)) >> 4

      def _chunk(c, carry):
        (t1a, t2a, t3a, i1a, i2a, i3a,
         t1b, t2b, t3b, i1b, i2b, i3b) = carry
        coff = pl.multiple_of(c << 4, 16)
        xs_c = xs_v[pl.ds(coff, L)]
        ys_c = ys_v[pl.ds(coff, L)]
        zs_c = zs_v[pl.ds(coff, L)]
        bs_c = bx_v[pl.ds(coff, L)]
        for t in range(L):
          xj = _splat(xs_c, t)
          yj = _splat(ys_c, t)
          zj = _splat(zs_c, t)
          bj = _splat(bs_c, t)
          jv = jnp.full((L,), coff + t, dtype=jnp.int32)
          dx = yxa - xj
          dy = yya - yj
          dz = yza - zj
          d = dx * dx + dy * dy + dz * dz
          d = jnp.where(bj == yba, d, _INF)
          t1a, t2a, t3a, i1a, i2a, i3a = _insert(
              d, jv, t1a, t2a, t3a, i1a, i2a, i3a)
          dx = yxb - xj
          dy = yyb - yj
          dz = yzb - zj
          d = dx * dx + dy * dy + dz * dz
          d = jnp.where(bj == ybb, d, _INF)
          t1b, t2b, t3b, i1b, i2b, i3b = _insert(
              d, jv, t1b, t2b, t3b, i1b, i2b, i3b)
        return (t1a, t2a, t3a, i1a, i2a, i3a,
                t1b, t2b, t3b, i1b, i2b, i3b)

      inf = jnp.full((L,), _INF, dtype=jnp.float32)
      zero = jnp.zeros((L,), jnp.int32)
      (t1a, t2a, t3a, i1a, i2a, i3a,
       t1b, t2b, t3b, i1b, i2b, i3b) = lax.fori_loop(
          c_lo, c_hi, _chunk,
          (inf, inf, inf, zero, zero, zero,
           inf, inf, inf, zero, zero, zero))

      def _weights(t1, t2, t3):
        w1 = 1.0 / jnp.maximum(t1, 1e-16)
        w2 = 1.0 / jnp.maximum(t2, 1e-16)
        w3 = 1.0 / jnp.maximum(t3, 1e-16)
        winv = 1.0 / (w1 + w2 + w3)
        return w1 * winv, w2 * winv, w3 * winv

      w1a, w2a, w3a = _weights(t1a, t2a, t3a)
      w1b, w2b, w3b = _weights(t1b, t2b, t3b)

      idx_v[pg, pl.ds(0, L)] = i1a
      idx_v[pg, pl.ds(L, L)] = i2a
      idx_v[pg, pl.ds(2 * L, L)] = i3a
      idx_v[pg, pl.ds(3 * L, L)] = i1b
      idx_v[pg, pl.ds(4 * L, L)] = i2b
      idx_v[pg, pl.ds(5 * L, L)] = i3b

      @pl.when(pg == 0)
      def _():
        pltpu.async_copy(xf_h.at[idx_v.at[pg]], rows_v.at[pg], sem0)

      @pl.when(pg == 1)
      def _():
        pltpu.async_copy(xf_h.at[idx_v.at[pg]], rows_v.at[pg], sem1)

      return w1a, w2a, w3a, w1b, w2b, w3b

    def _combine(p, w1a, w2a, w3a, w1b, w2b, w3b):
      pg = p & 1
      goff = p << 5

      @pl.when(pg == 0)
      def _():
        pltpu.make_async_copy(xf_h.at[idx_v.at[pg]], rows_v.at[pg],
                              sem0).wait()

      @pl.when(pg == 1)
      def _():
        pltpu.make_async_copy(xf_h.at[idx_v.at[pg]], rows_v.at[pg],
                              sem1).wait()

      def _rows(base_row, xi_off, w1, w2, w3):
        for r in range(L):
          a1 = w1[r]
          a2 = w2[r]
          a3 = w3[r]
          for q in range(C // L):
            qo = q * L
            acc = (rows_v[pg, base_row + r, pl.ds(qo, L)] * a1
                   + rows_v[pg, base_row + L + r, pl.ds(qo, L)] * a2
                   + rows_v[pg, base_row + 2 * L + r, pl.ds(qo, L)] * a3)
            xi_all[goff + xi_off + r, pl.ds(qo, L)] = acc

      _rows(0, 0, w1a, w2a, w3a)
      _rows(3 * L, L, w1b, w2b, w3b)

    w_first = _knn_start(jnp.int32(0))

    def _pipe(p, w_prev):
      w_cur = _knn_start(p)
      _combine(p - 1, *w_prev)
      return w_cur

    w_last = lax.fori_loop(1, NG // 2, _pipe, w_first)
    _combine(jnp.int32(NG // 2 - 1), *w_last)

    pltpu.sync_copy(xi_all, out_h.at[pl.ds(base, PF)])

  return body(xs, ys, zs, bx, fx, fy, fz, fb, xfeat)


def _mlp_tc(xi, xsk, W1a, W1b, b1, W2, b2, pe, Wp, bp):
  BLK = 512
  grid = (NY // BLK,)

  def body(xi_r, xsk_r, W1a_r, W1b_r, b1_r, W2_r, b2_r, pe_r, Wp_r, bp_r,
           out_r):
    h = xi_r[...] @ W1a_r[...] + xsk_r[...] @ W1b_r[...] + b1_r[...]
    h = jnp.maximum(h, 0.0)
    h = h @ W2_r[...] + b2_r[...]
    e = pl.program_id(0) // (2048 // BLK)
    per = pe_r[pl.ds(e, 1), :]
    pr = jnp.maximum(per @ Wp_r[...] + bp_r[...], 0.0)
    out_r[...] = h * pr

  full = lambda shape: pl.BlockSpec(shape, lambda i: (0, 0))
  return pl.pallas_call(
      body,
      grid=grid,
      in_specs=[
          pl.BlockSpec((BLK, C), lambda i: (i, 0)),
          pl.BlockSpec((BLK, 64), lambda i: (i, 0)),
          full((C, C)),
          full((64, C)),
          full((1, C)),
          full((C, C)),
          full((1, C)),
          full((NB, 512)),
          full((512, C)),
          full((1, C)),
      ],
      out_specs=pl.BlockSpec((BLK, C), lambda i: (i, 0)),
      out_shape=jax.ShapeDtypeStruct((NY, C), jnp.float32),
  )(xi, xsk, W1a, W1b, b1, W2, b2, pe, Wp, bp)


def kernel(par_embedding, x, pos, batch, x_skip, pos_skip, batch_skip,
           W1, b1, W2, b2, Wp, bp):
  xs = jnp.copy(pos[:, 0])
  ys = jnp.copy(pos[:, 1])
  zs = jnp.copy(pos[:, 2])
  bx = batch.astype(jnp.int32)
  fx = jnp.copy(pos_skip[:, 0])
  fy = jnp.copy(pos_skip[:, 1])
  fz = jnp.copy(pos_skip[:, 2])
  fb = batch_skip.astype(jnp.int32)

  xi = _knn_gather_sc(xs, ys, zs, bx, fx, fy, fz, fb, x)

  W1a = W1[:C]
  W1b = W1[C:]
  pe = par_embedding.reshape(NB, 512)
  out = _mlp_tc(xi, x_skip, W1a, W1b, b1.reshape(1, C), W2, b2.reshape(1, C),
                pe, Wp, bp.reshape(1, C))
  return (out, pos_skip, batch_skip)


# speedup vs baseline: 1.1111x; 1.1111x over previous
import functools

import jax
import jax.numpy as jnp
from jax import lax
from jax.experimental import pallas as pl
from jax.experimental.pallas import tpu as pltpu
from jax.experimental.pallas import tpu_sc as plsc

NC = 2
NS = 16
NW = NC * NS
L = 16

NX = 4096
NY = 16384
C = 128
NB = 8
PF = NY // NW
NG = PF // L
NCHUNK = NX // L

_INF = float("inf")


def _splat(v, t):
  return v[jnp.full((L,), t, dtype=jnp.int32)]


def _knn_gather_sc(xs, ys, zs, bx, fx, fy, fz, fb, xfeat):
  mesh = plsc.VectorSubcoreMesh(core_axis_name="c", subcore_axis_name="s")

  @functools.partial(
      pl.kernel,
      mesh=mesh,
      out_type=jax.ShapeDtypeStruct((NY, C), jnp.float32),
      scratch_types=[
          pltpu.VMEM((NX,), jnp.float32),
          pltpu.VMEM((NX,), jnp.float32),
          pltpu.VMEM((NX,), jnp.float32),
          pltpu.VMEM((NX,), jnp.int32),
          pltpu.VMEM((PF,), jnp.float32),
          pltpu.VMEM((PF,), jnp.float32),
          pltpu.VMEM((PF,), jnp.float32),
          pltpu.VMEM((PF,), jnp.int32),
          pltpu.SMEM((L,), jnp.int32),
          pltpu.VMEM((2, 6 * L), jnp.int32),
          pltpu.VMEM((2, 6 * L, C), jnp.float32),
          pltpu.VMEM((PF, C), jnp.float32),
          pltpu.SemaphoreType.DMA,
          pltpu.SemaphoreType.DMA,
      ],
  )
  def body(xs_h, ys_h, zs_h, bx_h, fx_h, fy_h, fz_h, fb_h, xf_h, out_h,
           xs_v, ys_v, zs_v, bx_v, fx_v, fy_v, fz_v, fb_v, st_s, idx_v,
           rows_v, xi_all, sem0, sem1):
    wid = lax.axis_index("s") * NC + lax.axis_index("c")
    base = pl.multiple_of(wid << 9, 512)

    pltpu.sync_copy(xs_h, xs_v)
    pltpu.sync_copy(ys_h, ys_v)
    pltpu.sync_copy(zs_h, zs_v)
    pltpu.sync_copy(bx_h, bx_v)
    pltpu.sync_copy(fx_h.at[pl.ds(base, PF)], fx_v)
    pltpu.sync_copy(fy_h.at[pl.ds(base, PF)], fy_v)
    pltpu.sync_copy(fz_h.at[pl.ds(base, PF)], fz_v)
    pltpu.sync_copy(fb_h.at[pl.ds(base, PF)], fb_v)

    iota = lax.iota(jnp.int32, L)

    def _hist(c, cnt):
      bs_c = bx_v[pl.ds(c << 4, L)]
      for t in range(L):
        s = _splat(bs_c, t)
        cnt = cnt + jnp.where(s < iota, 1, 0).astype(jnp.int32)
      return cnt

    starts = lax.fori_loop(0, NCHUNK, _hist, jnp.zeros((L,), jnp.int32))
    for b in range(NB + 1):
      st_s[b] = starts[b]

    def _insert(d, jv, t1, t2, t3, i1, i2, i3):
      c1 = d < t1
      c2 = d < t2
      c3 = d < t3
      nt3 = jnp.where(c2, t2, jnp.where(c3, d, t3))
      ni3 = jnp.where(c2, i2, jnp.where(c3, jv, i3))
      nt2 = jnp.where(c1, t1, jnp.where(c2, d, t2))
      ni2 = jnp.where(c1, i1, jnp.where(c2, jv, i2))
      nt1 = jnp.where(c1, d, t1)
      ni1 = jnp.where(c1, jv, i1)
      return nt1, nt2, nt3, ni1, ni2, ni3

    def _knn_start(p):
      pg = p & 1
      goff = pl.multiple_of(p << 5, 32)
      yxa = fx_v[pl.ds(goff, L)]
      yya = fy_v[pl.ds(goff, L)]
      yza = fz_v[pl.ds(goff, L)]
      yba = fb_v[pl.ds(goff, L)]
      yxb = fx_v[pl.ds(goff + L, L)]
      yyb = fy_v[pl.ds(goff + L, L)]
      yzb = fz_v[pl.ds(goff + L, L)]
      ybb = fb_v[pl.ds(goff + L, L)]

      b_lo = yba[0]
      b_hi = ybb[15]
      j_lo = st_s[b_lo]
      j_hi = st_s[b_hi + 1]
      c_lo = j_lo >> 4
      c_hi = (j_hi + (L - 1)) >> 4

      def _chunk(c, carry):
        (t1a, t2a, t3a, i1a, i2a, i3a,
         t1b, t2b, t3b, i1b, i2b, i3b) = carry
        coff = pl.multiple_of(c << 4, 16)
        xs_c = xs_v[pl.ds(coff, L)]
        ys_c = ys_v[pl.ds(coff, L)]
        zs_c = zs_v[pl.ds(coff, L)]
        bs_c = bx_v[pl.ds(coff, L)]
        for t in range(L):
          xj = _splat(xs_c, t)
          yj = _splat(ys_c, t)
          zj = _splat(zs_c, t)
          bj = _splat(bs_c, t)
          jv = jnp.full((L,), coff + t, dtype=jnp.int32)
          dx = yxa - xj
          dy = yya - yj
          dz = yza - zj
          d = dx * dx + dy * dy + dz * dz
          d = jnp.where(bj == yba, d, _INF)
          t1a, t2a, t3a, i1a, i2a, i3a = _insert(
              d, jv, t1a, t2a, t3a, i1a, i2a, i3a)
          dx = yxb - xj
          dy = yyb - yj
          dz = yzb - zj
          d = dx * dx + dy * dy + dz * dz
          d = jnp.where(bj == ybb, d, _INF)
          t1b, t2b, t3b, i1b, i2b, i3b = _insert(
              d, jv, t1b, t2b, t3b, i1b, i2b, i3b)
        return (t1a, t2a, t3a, i1a, i2a, i3a,
                t1b, t2b, t3b, i1b, i2b, i3b)

      inf = jnp.full((L,), _INF, dtype=jnp.float32)
      zero = jnp.zeros((L,), jnp.int32)
      (t1a, t2a, t3a, i1a, i2a, i3a,
       t1b, t2b, t3b, i1b, i2b, i3b) = lax.fori_loop(
          c_lo, c_hi, _chunk,
          (inf, inf, inf, zero, zero, zero,
           inf, inf, inf, zero, zero, zero))

      def _weights(t1, t2, t3):
        w1 = 1.0 / jnp.maximum(t1, 1e-16)
        w2 = 1.0 / jnp.maximum(t2, 1e-16)
        w3 = 1.0 / jnp.maximum(t3, 1e-16)
        winv = 1.0 / (w1 + w2 + w3)
        return w1 * winv, w2 * winv, w3 * winv

      w1a, w2a, w3a = _weights(t1a, t2a, t3a)
      w1b, w2b, w3b = _weights(t1b, t2b, t3b)

      idx_v[pg, pl.ds(0, L)] = i1a
      idx_v[pg, pl.ds(L, L)] = i2a
      idx_v[pg, pl.ds(2 * L, L)] = i3a
      idx_v[pg, pl.ds(3 * L, L)] = i1b
      idx_v[pg, pl.ds(4 * L, L)] = i2b
      idx_v[pg, pl.ds(5 * L, L)] = i3b

      @pl.when(pg == 0)
      def _():
        pltpu.async_copy(xf_h.at[idx_v.at[pg]], rows_v.at[pg], sem0)

      @pl.when(pg == 1)
      def _():
        pltpu.async_copy(xf_h.at[idx_v.at[pg]], rows_v.at[pg], sem1)

      return w1a, w2a, w3a, w1b, w2b, w3b

    def _combine(p, w1a, w2a, w3a, w1b, w2b, w3b):
      pg = p & 1
      goff = p << 5

      @pl.when(pg == 0)
      def _():
        pltpu.make_async_copy(xf_h.at[idx_v.at[pg]], rows_v.at[pg],
                              sem0).wait()

      @pl.when(pg == 1)
      def _():
        pltpu.make_async_copy(xf_h.at[idx_v.at[pg]], rows_v.at[pg],
                              sem1).wait()

      def _rows(base_row, xi_off, w1, w2, w3):
        def _row(r, _):
          a1 = _splat(w1, r)
          a2 = _splat(w2, r)
          a3 = _splat(w3, r)
          for q in range(C // L):
            qo = q * L
            acc = (rows_v[pg, base_row + r, pl.ds(qo, L)] * a1
                   + rows_v[pg, base_row + L + r, pl.ds(qo, L)] * a2
                   + rows_v[pg, base_row + 2 * L + r, pl.ds(qo, L)] * a3)
            xi_all[goff + xi_off + r, pl.ds(qo, L)] = acc
          return 0
        lax.fori_loop(0, L, _row, 0)

      _rows(0, 0, w1a, w2a, w3a)
      _rows(3 * L, L, w1b, w2b, w3b)

    w_first = _knn_start(jnp.int32(0))

    def _pipe(p, w_prev):
      w_cur = _knn_start(p)
      _combine(p - 1, *w_prev)
      return w_cur

    w_last = lax.fori_loop(1, NG // 2, _pipe, w_first)
    _combine(jnp.int32(NG // 2 - 1), *w_last)

    pltpu.sync_copy(xi_all, out_h.at[pl.ds(base, PF)])

  return body(xs, ys, zs, bx, fx, fy, fz, fb, xfeat)


def _mlp_tc(xi, xsk, W1a, W1b, b1, W2, b2, pe, Wp, bp):
  BLK = 512
  grid = (NY // BLK,)

  def body(xi_r, xsk_r, W1a_r, W1b_r, b1_r, W2_r, b2_r, pe_r, Wp_r, bp_r,
           out_r):
    h = xi_r[...] @ W1a_r[...] + xsk_r[...] @ W1b_r[...] + b1_r[...]
    h = jnp.maximum(h, 0.0)
    h = h @ W2_r[...] + b2_r[...]
    e = pl.program_id(0) // (2048 // BLK)
    per = pe_r[pl.ds(e, 1), :]
    pr = jnp.maximum(per @ Wp_r[...] + bp_r[...], 0.0)
    out_r[...] = h * pr

  full = lambda shape: pl.BlockSpec(shape, lambda i: (0, 0))
  return pl.pallas_call(
      body,
      grid=grid,
      in_specs=[
          pl.BlockSpec((BLK, C), lambda i: (i, 0)),
          pl.BlockSpec((BLK, 64), lambda i: (i, 0)),
          full((C, C)),
          full((64, C)),
          full((1, C)),
          full((C, C)),
          full((1, C)),
          full((NB, 512)),
          full((512, C)),
          full((1, C)),
      ],
      out_specs=pl.BlockSpec((BLK, C), lambda i: (i, 0)),
      out_shape=jax.ShapeDtypeStruct((NY, C), jnp.float32),
  )(xi, xsk, W1a, W1b, b1, W2, b2, pe, Wp, bp)


def kernel(par_embedding, x, pos, batch, x_skip, pos_skip, batch_skip,
           W1, b1, W2, b2, Wp, bp):
  xs = jnp.copy(pos[:, 0])
  ys = jnp.copy(pos[:, 1])
  zs = jnp.copy(pos[:, 2])
  bx = batch.astype(jnp.int32)
  fx = jnp.copy(pos_skip[:, 0])
  fy = jnp.copy(pos_skip[:, 1])
  fz = jnp.copy(pos_skip[:, 2])
  fb = batch_skip.astype(jnp.int32)

  xi = _knn_gather_sc(xs, ys, zs, bx, fx, fy, fz, fb, x)

  W1a = W1[:C]
  W1b = W1[C:]
  pe = par_embedding.reshape(NB, 512)
  out = _mlp_tc(xi, x_skip, W1a, W1b, b1.reshape(1, C), W2, b2.reshape(1, C),
                pe, Wp, bp.reshape(1, C))
  return (out, pos_skip, batch_skip)
